# Initial kernel scaffold; baseline (speedup 1.0000x reference)
import jax
import jax.numpy as jnp
from jax.experimental import pallas as pl


def kernel(x, edge_index, W1, b1, W2, b2, Wl1, bl1, Wl2, bl2):
    raise NotImplementedError("write your pallas kernel here")


# speedup vs baseline: 8.2867x; 8.2867x over previous
import functools

import jax
import jax.numpy as jnp
from jax import lax
from jax.experimental import pallas as pl
from jax.experimental.pallas import tpu as pltpu
from jax.experimental.pallas import tpu_sc as plsc

NW = 32
CHUNK = 128
_SC_PARAMS = pltpu.CompilerParams(needs_layout_passes=False)


def _matmul(x, w, mb):
    m, k = x.shape
    _, n = w.shape

    def body(x_ref, w_ref, o_ref):
        o_ref[...] = jnp.dot(x_ref[...], w_ref[...],
                             preferred_element_type=jnp.float32)

    return pl.pallas_call(
        body,
        grid=(m // mb,),
        in_specs=[pl.BlockSpec((mb, k), lambda i: (i, 0)),
                  pl.BlockSpec((k, n), lambda i: (0, 0))],
        out_specs=pl.BlockSpec((mb, n), lambda i: (i, 0)),
        out_shape=jax.ShapeDtypeStruct((m, n), jnp.float32),
    )(x, w)


def _dinv_scale(degp_t, xw):
    n, p = degp_t.shape
    h = xw.shape[1]
    mb = 1000

    def body(d_ref, x_ref, y_ref, dinv_ref):
        deg = jnp.sum(d_ref[...], axis=1).astype(jnp.float32) + 1.0
        dinv = lax.rsqrt(deg)
        y_ref[...] = x_ref[...] * dinv[:, None]
        dinv_ref[...] = dinv[:, None]

    return pl.pallas_call(
        body,
        grid=(n // mb,),
        in_specs=[pl.BlockSpec((mb, p), lambda i: (i, 0)),
                  pl.BlockSpec((mb, h), lambda i: (i, 0))],
        out_specs=[pl.BlockSpec((mb, h), lambda i: (i, 0)),
                   pl.BlockSpec((mb, 1), lambda i: (i, 0))],
        out_shape=[jax.ShapeDtypeStruct((n, h), jnp.float32),
                   jax.ShapeDtypeStruct((n, 1), jnp.float32)],
    )(degp_t, xw)


def _layer_mm(sp, y, dinv, b, w):
    _, n, h = sp.shape
    mb = 1000

    def body(sp_ref, y_ref, di_ref, b_ref, w_ref, o_ref):
        t = di_ref[...] * (sp_ref[0] + sp_ref[1] + y_ref[...]) + b_ref[...]
        hh = jnp.maximum(t, 0.0)
        o_ref[...] = jnp.dot(hh, w_ref[...],
                             preferred_element_type=jnp.float32) * di_ref[...]

    return pl.pallas_call(
        body,
        grid=(n // mb,),
        in_specs=[pl.BlockSpec((2, mb, h), lambda i: (0, i, 0)),
                  pl.BlockSpec((mb, h), lambda i: (i, 0)),
                  pl.BlockSpec((mb, 1), lambda i: (i, 0)),
                  pl.BlockSpec((h,), lambda i: (0,)),
                  pl.BlockSpec((h, h), lambda i: (0, 0))],
        out_specs=pl.BlockSpec((mb, h), lambda i: (i, 0)),
        out_shape=jax.ShapeDtypeStruct((n, h), jnp.float32),
    )(sp, y, dinv, b, w)


def _layer_mm_final(sp, y, dinv, b, wcat, bl1):
    _, n, h = sp.shape
    mb = 1000

    def body(sp_ref, y_ref, di_ref, b_ref, w_ref, bl1_ref, a_ref, bt_ref):
        t = di_ref[...] * (sp_ref[0] + sp_ref[1] + y_ref[...]) + b_ref[...]
        hh = jnp.maximum(t, 0.0)
        acc = jnp.dot(hh, w_ref[...], preferred_element_type=jnp.float32)
        a_ref[...] = acc[:, :h] + bl1_ref[...]
        bt_ref[...] = acc[:, h:]

    return pl.pallas_call(
        body,
        grid=(n // mb,),
        in_specs=[pl.BlockSpec((2, mb, h), lambda i: (0, i, 0)),
                  pl.BlockSpec((mb, h), lambda i: (i, 0)),
                  pl.BlockSpec((mb, 1), lambda i: (i, 0)),
                  pl.BlockSpec((h,), lambda i: (0,)),
                  pl.BlockSpec((h, 2 * h), lambda i: (0, 0)),
                  pl.BlockSpec((h,), lambda i: (0,))],
        out_specs=[pl.BlockSpec((mb, h), lambda i: (i, 0)),
                   pl.BlockSpec((mb, h), lambda i: (i, 0))],
        out_shape=[jax.ShapeDtypeStruct((n, h), jnp.float32),
                   jax.ShapeDtypeStruct((n, h), jnp.float32)],
    )(sp, y, dinv, b, wcat, bl1)


def _final(z, wl2, bl2):
    e, h = z.shape
    c = wl2.shape[1]
    mb = 4000

    def body(z_ref, w_ref, b_ref, o_ref):
        zz = jnp.maximum(z_ref[...], 0.0)
        l = jnp.dot(zz, w_ref[...], preferred_element_type=jnp.float32) + b_ref[...]
        m = jnp.max(l, axis=1, keepdims=True)
        s = l - m
        lse = jnp.log(jnp.sum(jnp.exp(s), axis=1, keepdims=True))
        o_ref[...] = s - lse

    return pl.pallas_call(
        body,
        grid=(e // mb,),
        in_specs=[pl.BlockSpec((mb, h), lambda i: (i, 0)),
                  pl.BlockSpec((h, c), lambda i: (0, 0)),
                  pl.BlockSpec((c,), lambda i: (0,))],
        out_specs=pl.BlockSpec((mb, c), lambda i: (i, 0)),
        out_shape=jax.ShapeDtypeStruct((e, c), jnp.float32),
    )(z, wl2, bl2)


def _sc_degree(dst, n):
    e = dst.shape[0]
    epw = e // NW
    full = epw // 16
    rem = epw - full * 16
    mesh = plsc.VectorSubcoreMesh(core_axis_name="c", subcore_axis_name="s")

    @functools.partial(
        pl.kernel,
        out_type=jax.ShapeDtypeStruct((NW, n), jnp.float32),
        mesh=mesh,
        compiler_params=_SC_PARAMS,
        scratch_types=[
            pltpu.VMEM((epw + 16,), jnp.int32),
            pltpu.VMEM((n,), jnp.float32),
        ],
    )
    def k(dst_hbm, out_hbm, idx_v, hist_v):
        cid = lax.axis_index("c")
        sid = lax.axis_index("s")
        wid = sid * 2 + cid
        zeros16 = jnp.zeros((16,), jnp.float32)
        ones16 = jnp.ones((16,), jnp.float32)

        def zero_body(i, _):
            hist_v[pl.ds(i * 16, 16)] = zeros16
            return 0
        lax.fori_loop(0, n // 16, zero_body, 0)

        pltpu.sync_copy(dst_hbm.at[pl.ds(wid * epw, epw)], idx_v.at[pl.ds(0, epw)])

        def body(i, _):
            v = idx_v[pl.ds(i * 16, 16)]
            plsc.addupdate_scatter(hist_v, [v], ones16)
            return 0
        lax.fori_loop(0, full, body, 0)
        if rem:
            v = idx_v[pl.ds(full * 16, 16)]
            mask = lax.iota(jnp.int32, 16) < rem
            plsc.addupdate_scatter(hist_v, [v], ones16, mask=mask)

        pltpu.sync_copy(hist_v, out_hbm.at[wid])

    return k(dst)


def _sc_aggregate(y, src, dst, zeros):
    n, h = y.shape
    e = src.shape[0]
    epw = e // NW
    nfull = epw // CHUNK
    tail = epw - nfull * CHUNK
    rps = (n // (16 * 8)) * 8
    rextra = n - 16 * rps
    mesh = plsc.VectorSubcoreMesh(core_axis_name="c", subcore_axis_name="s")

    @functools.partial(
        pl.kernel,
        out_type=jax.ShapeDtypeStruct((2, n, h), jnp.float32),
        mesh=mesh,
        compiler_params=_SC_PARAMS,
        scratch_types=[
            pltpu.VMEM((CHUNK,), jnp.int32),
            pltpu.VMEM((CHUNK,), jnp.int32),
            pltpu.VMEM((CHUNK, h), jnp.float32),
            pltpu.VMEM((tail,), jnp.int32),
            pltpu.VMEM((tail,), jnp.int32),
            pltpu.VMEM((tail, h), jnp.float32),
            pltpu.VMEM_SHARED((n, h), jnp.float32),
            pltpu.SemaphoreType.DMA,
        ],
    )
    def k(y_hbm, src_hbm, dst_hbm, zero_hbm, out_hbm,
          sidx, didx, rows, sidx_t, didx_t, rows_t, stab, sem):
        cid = lax.axis_index("c")
        sid = lax.axis_index("s")
        wid = sid * 2 + cid
        base = wid * epw
        r0 = pl.multiple_of(sid * rps, 8)
        pltpu.sync_copy(zero_hbm.at[pl.ds(r0, rps)], stab.at[pl.ds(r0, rps)])
        if rextra:
            @pl.when(sid == 15)
            def _():
                pltpu.sync_copy(zero_hbm.at[pl.ds(16 * rps, rextra)],
                                stab.at[pl.ds(16 * rps, rextra)])
        plsc.subcore_barrier()

        def chunk(j, _):
            off = base + j * CHUNK
            pltpu.sync_copy(src_hbm.at[pl.ds(off, CHUNK)], sidx)
            pltpu.sync_copy(dst_hbm.at[pl.ds(off, CHUNK)], didx)
            pltpu.async_copy(y_hbm.at[sidx], rows, sem).wait()
            pltpu.sync_copy(rows, stab.at[didx], add=True)
            return 0
        lax.fori_loop(0, nfull, chunk, 0)
        if tail:
            off = base + nfull * CHUNK
            pltpu.sync_copy(src_hbm.at[pl.ds(off, tail)], sidx_t)
            pltpu.sync_copy(dst_hbm.at[pl.ds(off, tail)], didx_t)
            pltpu.async_copy(y_hbm.at[sidx_t], rows_t, sem).wait()
            pltpu.sync_copy(rows_t, stab.at[didx_t], add=True)

        plsc.subcore_barrier()
        pltpu.sync_copy(stab.at[pl.ds(r0, rps)],
                        out_hbm.at[cid, pl.ds(r0, rps)])
        if rextra:
            @pl.when(sid == 15)
            def _():
                pltpu.sync_copy(stab.at[pl.ds(16 * rps, rextra)],
                                out_hbm.at[cid, pl.ds(16 * rps, rextra)])

    return k(y, src, dst, zeros)


def _sc_edge(a_t, b_t, src, dst):
    n, h = a_t.shape
    e = src.shape[0]
    epw = e // NW
    nfull = epw // CHUNK
    tail = epw - nfull * CHUNK
    mesh = plsc.VectorSubcoreMesh(core_axis_name="c", subcore_axis_name="s")

    @functools.partial(
        pl.kernel,
        out_type=jax.ShapeDtypeStruct((e, h), jnp.float32),
        mesh=mesh,
        compiler_params=_SC_PARAMS,
        scratch_types=[
            pltpu.VMEM((CHUNK,), jnp.int32),
            pltpu.VMEM((CHUNK,), jnp.int32),
            pltpu.VMEM((CHUNK, h), jnp.float32),
            pltpu.VMEM((CHUNK, h), jnp.float32),
            pltpu.VMEM((tail,), jnp.int32),
            pltpu.VMEM((tail,), jnp.int32),
            pltpu.VMEM((tail, h), jnp.float32),
            pltpu.VMEM((tail, h), jnp.float32),
            pltpu.SemaphoreType.DMA,
            pltpu.SemaphoreType.DMA,
        ],
    )
    def k(a_hbm, b_hbm, src_hbm, dst_hbm, z_hbm,
          sidx, didx, ra, rb, sidx_t, didx_t, ra_t, rb_t, sema, semb):
        cid = lax.axis_index("c")
        sid = lax.axis_index("s")
        wid = sid * 2 + cid
        base = wid * epw

        def do_chunk(off, m, si, di, va, vb):
            pltpu.sync_copy(src_hbm.at[pl.ds(off, m)], si)
            pltpu.sync_copy(dst_hbm.at[pl.ds(off, m)], di)
            ca = pltpu.async_copy(a_hbm.at[si], va, sema)
            cb = pltpu.async_copy(b_hbm.at[di], vb, semb)
            ca.wait()
            cb.wait()

            def addrow(r, _):
                for c in range(h // 16):
                    plsc.addupdate(va.at[r, pl.ds(c * 16, 16)],
                                   vb[r, pl.ds(c * 16, 16)])
                return 0
            lax.fori_loop(0, m, addrow, 0)
            pltpu.sync_copy(va, z_hbm.at[pl.ds(off, m)])

        def chunk(j, _):
            do_chunk(pl.multiple_of(base + j * CHUNK, 8), CHUNK,
                     sidx, didx, ra, rb)
            return 0
        lax.fori_loop(0, nfull, chunk, 0)
        if tail:
            do_chunk(pl.multiple_of(base + nfull * CHUNK, 8), tail,
                     sidx_t, didx_t, ra_t, rb_t)

    return k(a_t, b_t, src, dst)


def kernel(x, edge_index, W1, b1, W2, b2, Wl1, bl1, Wl2, bl2):
    n, _ = x.shape
    h = W1.shape[1]
    src = edge_index[0]
    dst = edge_index[1]

    xw1 = _matmul(x, W1, 1000)
    degp = _sc_degree(dst, n)
    y1, dinv = _dinv_scale(degp.T, xw1)
    zeros = jnp.zeros((n, h), jnp.float32)
    sp1 = _sc_aggregate(y1, src, dst, zeros)
    y2 = _layer_mm(sp1, y1, dinv, b1, W2)
    sp2 = _sc_aggregate(y2, src, dst, zeros)
    wcat = jnp.concatenate([Wl1[:h], Wl1[h:]], axis=1)
    a_t, b_t = _layer_mm_final(sp2, y2, dinv, b2, wcat, bl1)
    z = _sc_edge(a_t, b_t, src, dst)
    return _final(z, Wl2, bl2)


# speedup vs baseline: 9.5678x; 1.1546x over previous
import functools

import jax
import jax.numpy as jnp
from jax import lax
from jax.experimental import pallas as pl
from jax.experimental.pallas import tpu as pltpu
from jax.experimental.pallas import tpu_sc as plsc

NW = 32
CHUNK = 128
_SC_PARAMS = pltpu.CompilerParams(needs_layout_passes=False)


def _matmul(x, w, mb):
    m, k = x.shape
    _, n = w.shape

    def body(x_ref, w_ref, o_ref):
        o_ref[...] = jnp.dot(x_ref[...], w_ref[...],
                             preferred_element_type=jnp.float32)

    return pl.pallas_call(
        body,
        grid=(m // mb,),
        in_specs=[pl.BlockSpec((mb, k), lambda i: (i, 0)),
                  pl.BlockSpec((k, n), lambda i: (0, 0))],
        out_specs=pl.BlockSpec((mb, n), lambda i: (i, 0)),
        out_shape=jax.ShapeDtypeStruct((m, n), jnp.float32),
    )(x, w)


def _dinv_scale(degp_t, xw):
    n, p = degp_t.shape
    h = xw.shape[1]
    mb = 1000

    def body(d_ref, x_ref, y_ref, dinv_ref):
        deg = jnp.sum(d_ref[...], axis=1).astype(jnp.float32) + 1.0
        dinv = lax.rsqrt(deg)
        y_ref[...] = x_ref[...] * dinv[:, None]
        dinv_ref[...] = dinv[:, None]

    return pl.pallas_call(
        body,
        grid=(n // mb,),
        in_specs=[pl.BlockSpec((mb, p), lambda i: (i, 0)),
                  pl.BlockSpec((mb, h), lambda i: (i, 0))],
        out_specs=[pl.BlockSpec((mb, h), lambda i: (i, 0)),
                   pl.BlockSpec((mb, 1), lambda i: (i, 0))],
        out_shape=[jax.ShapeDtypeStruct((n, h), jnp.float32),
                   jax.ShapeDtypeStruct((n, 1), jnp.float32)],
    )(degp_t, xw)


def _layer_mm(sp, y, dinv, b, w):
    _, n, h = sp.shape
    mb = 1000

    def body(sp_ref, y_ref, di_ref, b_ref, w_ref, o_ref):
        t = di_ref[...] * (sp_ref[0] + sp_ref[1] + y_ref[...]) + b_ref[...]
        hh = jnp.maximum(t, 0.0)
        o_ref[...] = jnp.dot(hh, w_ref[...],
                             preferred_element_type=jnp.float32) * di_ref[...]

    return pl.pallas_call(
        body,
        grid=(n // mb,),
        in_specs=[pl.BlockSpec((2, mb, h), lambda i: (0, i, 0)),
                  pl.BlockSpec((mb, h), lambda i: (i, 0)),
                  pl.BlockSpec((mb, 1), lambda i: (i, 0)),
                  pl.BlockSpec((h,), lambda i: (0,)),
                  pl.BlockSpec((h, h), lambda i: (0, 0))],
        out_specs=pl.BlockSpec((mb, h), lambda i: (i, 0)),
        out_shape=jax.ShapeDtypeStruct((n, h), jnp.float32),
    )(sp, y, dinv, b, w)


def _layer_mm_final(sp, y, dinv, b, wcat, bl1):
    _, n, h = sp.shape
    mb = 1000

    def body(sp_ref, y_ref, di_ref, b_ref, w_ref, bl1_ref, a_ref, bt_ref):
        t = di_ref[...] * (sp_ref[0] + sp_ref[1] + y_ref[...]) + b_ref[...]
        hh = jnp.maximum(t, 0.0)
        acc = jnp.dot(hh, w_ref[...], preferred_element_type=jnp.float32)
        a_ref[...] = acc[:, :h] + bl1_ref[...]
        bt_ref[...] = acc[:, h:]

    return pl.pallas_call(
        body,
        grid=(n // mb,),
        in_specs=[pl.BlockSpec((2, mb, h), lambda i: (0, i, 0)),
                  pl.BlockSpec((mb, h), lambda i: (i, 0)),
                  pl.BlockSpec((mb, 1), lambda i: (i, 0)),
                  pl.BlockSpec((h,), lambda i: (0,)),
                  pl.BlockSpec((h, 2 * h), lambda i: (0, 0)),
                  pl.BlockSpec((h,), lambda i: (0,))],
        out_specs=[pl.BlockSpec((mb, h), lambda i: (i, 0)),
                   pl.BlockSpec((mb, h), lambda i: (i, 0))],
        out_shape=[jax.ShapeDtypeStruct((n, h), jnp.float32),
                   jax.ShapeDtypeStruct((n, h), jnp.float32)],
    )(sp, y, dinv, b, wcat, bl1)


def _final(z, wl2, bl2):
    e, h = z.shape
    c = wl2.shape[1]
    mb = 4000

    def body(z_ref, w_ref, b_ref, o_ref):
        zz = jnp.maximum(z_ref[...], 0.0)
        l = jnp.dot(zz, w_ref[...], preferred_element_type=jnp.float32) + b_ref[...]
        m = jnp.max(l, axis=1, keepdims=True)
        s = l - m
        lse = jnp.log(jnp.sum(jnp.exp(s), axis=1, keepdims=True))
        o_ref[...] = s - lse

    return pl.pallas_call(
        body,
        grid=(e // mb,),
        in_specs=[pl.BlockSpec((mb, h), lambda i: (i, 0)),
                  pl.BlockSpec((h, c), lambda i: (0, 0)),
                  pl.BlockSpec((c,), lambda i: (0,))],
        out_specs=pl.BlockSpec((mb, c), lambda i: (i, 0)),
        out_shape=jax.ShapeDtypeStruct((e, c), jnp.float32),
    )(z, wl2, bl2)


def _sc_degree(dst, n):
    e = dst.shape[0]
    epw = e // NW
    full = epw // 16
    rem = epw - full * 16
    mesh = plsc.VectorSubcoreMesh(core_axis_name="c", subcore_axis_name="s")

    @functools.partial(
        pl.kernel,
        out_type=jax.ShapeDtypeStruct((NW, n), jnp.float32),
        mesh=mesh,
        compiler_params=_SC_PARAMS,
        scratch_types=[
            pltpu.VMEM((epw + 16,), jnp.int32),
            pltpu.VMEM((n,), jnp.float32),
        ],
    )
    def k(dst_hbm, out_hbm, idx_v, hist_v):
        cid = lax.axis_index("c")
        sid = lax.axis_index("s")
        wid = sid * 2 + cid
        zeros16 = jnp.zeros((16,), jnp.float32)
        ones16 = jnp.ones((16,), jnp.float32)

        def zero_body(i, _):
            hist_v[pl.ds(i * 16, 16)] = zeros16
            return 0
        lax.fori_loop(0, n // 16, zero_body, 0)

        pltpu.sync_copy(dst_hbm.at[pl.ds(wid * epw, epw)], idx_v.at[pl.ds(0, epw)])

        def body(i, _):
            v = idx_v[pl.ds(i * 16, 16)]
            plsc.addupdate_scatter(hist_v, [v], ones16)
            return 0
        lax.fori_loop(0, full, body, 0)
        if rem:
            v = idx_v[pl.ds(full * 16, 16)]
            mask = lax.iota(jnp.int32, 16) < rem
            plsc.addupdate_scatter(hist_v, [v], ones16, mask=mask)

        pltpu.sync_copy(hist_v, out_hbm.at[wid])

    return k(dst)


def _sc_aggregate(y, src, dst, zeros):
    n, h = y.shape
    e = src.shape[0]
    epw = e // NW
    nfull = epw // CHUNK
    tail = epw - nfull * CHUNK
    rps = (n // (16 * 8)) * 8
    rextra = n - 16 * rps
    mesh = plsc.VectorSubcoreMesh(core_axis_name="c", subcore_axis_name="s")

    npairs = nfull // 2
    leftover = nfull - 2 * npairs

    @functools.partial(
        pl.kernel,
        out_type=jax.ShapeDtypeStruct((2, n, h), jnp.float32),
        mesh=mesh,
        compiler_params=_SC_PARAMS,
        scratch_types=[
            pltpu.VMEM((epw + 16,), jnp.int32),
            pltpu.VMEM((epw + 16,), jnp.int32),
            pltpu.VMEM((CHUNK,), jnp.int32),
            pltpu.VMEM((CHUNK,), jnp.int32),
            pltpu.VMEM((CHUNK,), jnp.int32),
            pltpu.VMEM((CHUNK,), jnp.int32),
            pltpu.VMEM((CHUNK, h), jnp.float32),
            pltpu.VMEM((CHUNK, h), jnp.float32),
            pltpu.VMEM((max(tail, 1),), jnp.int32),
            pltpu.VMEM((max(tail, 1),), jnp.int32),
            pltpu.VMEM((max(tail, 1), h), jnp.float32),
            pltpu.VMEM_SHARED((n, h), jnp.float32),
            pltpu.SemaphoreType.DMA,
            pltpu.SemaphoreType.DMA,
            pltpu.SemaphoreType.DMA,
            pltpu.SemaphoreType.DMA,
        ],
    )
    def k(y_hbm, src_hbm, dst_hbm, zero_hbm, out_hbm,
          sall, dall, si0, si1, di0, di1, rows0, rows1,
          sidx_t, didx_t, rows_t, stab,
          semg0, semg1, sems0, sems1):
        cid = lax.axis_index("c")
        sid = lax.axis_index("s")
        wid = sid * 2 + cid
        base = wid * epw
        r0 = pl.multiple_of(sid * rps, 8)
        pltpu.sync_copy(zero_hbm.at[pl.ds(r0, rps)], stab.at[pl.ds(r0, rps)])
        if rextra:
            @pl.when(sid == 15)
            def _():
                pltpu.sync_copy(zero_hbm.at[pl.ds(16 * rps, rextra)],
                                stab.at[pl.ds(16 * rps, rextra)])
        pltpu.sync_copy(src_hbm.at[pl.ds(base, epw)], sall.at[pl.ds(0, epw)])
        pltpu.sync_copy(dst_hbm.at[pl.ds(base, epw)], dall.at[pl.ds(0, epw)])
        plsc.subcore_barrier()

        def stage(j, buf_all, buf_idx, m):
            for c in range(m // 16):
                buf_idx[pl.ds(c * 16, 16)] = buf_all[pl.ds(j * CHUNK + c * 16, 16)]

        def pair(t, _):
            a = 2 * t
            b = a + 1
            stage(a, sall, si0, CHUNK)
            stage(a, dall, di0, CHUNK)
            stage(b, sall, si1, CHUNK)
            stage(b, dall, di1, CHUNK)
            ga = pltpu.async_copy(y_hbm.at[si0], rows0, semg0)
            gb = pltpu.async_copy(y_hbm.at[si1], rows1, semg1)
            ga.wait()
            sa = pltpu.async_copy(rows0, stab.at[di0], sems0, add=True)
            gb.wait()
            sb = pltpu.async_copy(rows1, stab.at[di1], sems1, add=True)
            sa.wait()
            sb.wait()
            return 0
        lax.fori_loop(0, npairs, pair, 0)

        if leftover:
            j = 2 * npairs
            stage(j, sall, si0, CHUNK)
            stage(j, dall, di0, CHUNK)
            pltpu.async_copy(y_hbm.at[si0], rows0, semg0).wait()
            pltpu.async_copy(rows0, stab.at[di0], sems0, add=True).wait()
        if tail:
            off = base + nfull * CHUNK
            pltpu.sync_copy(src_hbm.at[pl.ds(off, tail)], sidx_t)
            pltpu.sync_copy(dst_hbm.at[pl.ds(off, tail)], didx_t)
            pltpu.async_copy(y_hbm.at[sidx_t], rows_t, semg1).wait()
            pltpu.async_copy(rows_t, stab.at[didx_t], sems1, add=True).wait()

        plsc.subcore_barrier()
        pltpu.sync_copy(stab.at[pl.ds(r0, rps)],
                        out_hbm.at[cid, pl.ds(r0, rps)])
        if rextra:
            @pl.when(sid == 15)
            def _():
                pltpu.sync_copy(stab.at[pl.ds(16 * rps, rextra)],
                                out_hbm.at[cid, pl.ds(16 * rps, rextra)])

    return k(y, src, dst, zeros)


def _sc_edge(a_t, b_t, src, dst):
    n, h = a_t.shape
    e = src.shape[0]
    epw = e // NW
    nfull = epw // CHUNK
    tail = epw - nfull * CHUNK
    mesh = plsc.VectorSubcoreMesh(core_axis_name="c", subcore_axis_name="s")

    @functools.partial(
        pl.kernel,
        out_type=jax.ShapeDtypeStruct((e, h), jnp.float32),
        mesh=mesh,
        compiler_params=_SC_PARAMS,
        scratch_types=[
            pltpu.VMEM((CHUNK,), jnp.int32),
            pltpu.VMEM((CHUNK,), jnp.int32),
            pltpu.VMEM((CHUNK, h), jnp.float32),
            pltpu.VMEM((CHUNK, h), jnp.float32),
            pltpu.VMEM((tail,), jnp.int32),
            pltpu.VMEM((tail,), jnp.int32),
            pltpu.VMEM((tail, h), jnp.float32),
            pltpu.VMEM((tail, h), jnp.float32),
            pltpu.SemaphoreType.DMA,
            pltpu.SemaphoreType.DMA,
        ],
    )
    def k(a_hbm, b_hbm, src_hbm, dst_hbm, z_hbm,
          sidx, didx, ra, rb, sidx_t, didx_t, ra_t, rb_t, sema, semb):
        cid = lax.axis_index("c")
        sid = lax.axis_index("s")
        wid = sid * 2 + cid
        base = wid * epw

        def do_chunk(off, m, si, di, va, vb):
            pltpu.sync_copy(src_hbm.at[pl.ds(off, m)], si)
            pltpu.sync_copy(dst_hbm.at[pl.ds(off, m)], di)
            ca = pltpu.async_copy(a_hbm.at[si], va, sema)
            cb = pltpu.async_copy(b_hbm.at[di], vb, semb)
            ca.wait()
            cb.wait()

            def addrow(r, _):
                for c in range(h // 16):
                    plsc.addupdate(va.at[r, pl.ds(c * 16, 16)],
                                   vb[r, pl.ds(c * 16, 16)])
                return 0
            lax.fori_loop(0, m, addrow, 0)
            pltpu.sync_copy(va, z_hbm.at[pl.ds(off, m)])

        def chunk(j, _):
            do_chunk(pl.multiple_of(base + j * CHUNK, 8), CHUNK,
                     sidx, didx, ra, rb)
            return 0
        lax.fori_loop(0, nfull, chunk, 0)
        if tail:
            do_chunk(pl.multiple_of(base + nfull * CHUNK, 8), tail,
                     sidx_t, didx_t, ra_t, rb_t)

    return k(a_t, b_t, src, dst)


def kernel(x, edge_index, W1, b1, W2, b2, Wl1, bl1, Wl2, bl2):
    n, _ = x.shape
    h = W1.shape[1]
    src = edge_index[0]
    dst = edge_index[1]

    xw1 = _matmul(x, W1, 1000)
    degp = _sc_degree(dst, n)
    y1, dinv = _dinv_scale(degp.T, xw1)
    zeros = jnp.zeros((n, h), jnp.float32)
    sp1 = _sc_aggregate(y1, src, dst, zeros)
    y2 = _layer_mm(sp1, y1, dinv, b1, W2)
    sp2 = _sc_aggregate(y2, src, dst, zeros)
    wcat = jnp.concatenate([Wl1[:h], Wl1[h:]], axis=1)
    a_t, b_t = _layer_mm_final(sp2, y2, dinv, b2, wcat, bl1)
    z = _sc_edge(a_t, b_t, src, dst)
    return _final(z, Wl2, bl2)


# speedup vs baseline: 10.6323x; 1.1113x over previous
import functools

import jax
import jax.numpy as jnp
from jax import lax
from jax.experimental import pallas as pl
from jax.experimental.pallas import tpu as pltpu
from jax.experimental.pallas import tpu_sc as plsc

NW = 32
CHUNK = 128
_SC_PARAMS = pltpu.CompilerParams(needs_layout_passes=False)


def _matmul(x, w, mb):
    m, k = x.shape
    _, n = w.shape

    def body(x_ref, w_ref, o_ref):
        o_ref[...] = jnp.dot(x_ref[...], w_ref[...],
                             preferred_element_type=jnp.float32)

    return pl.pallas_call(
        body,
        grid=(m // mb,),
        in_specs=[pl.BlockSpec((mb, k), lambda i: (i, 0)),
                  pl.BlockSpec((k, n), lambda i: (0, 0))],
        out_specs=pl.BlockSpec((mb, n), lambda i: (i, 0)),
        out_shape=jax.ShapeDtypeStruct((m, n), jnp.float32),
    )(x, w)


def _dinv_scale(degp_t, xw):
    n, p = degp_t.shape
    h = xw.shape[1]
    mb = 1000

    def body(d_ref, x_ref, y_ref, dinv_ref):
        deg = jnp.sum(d_ref[...], axis=1).astype(jnp.float32) + 1.0
        dinv = lax.rsqrt(deg)
        y_ref[...] = x_ref[...] * dinv[:, None]
        dinv_ref[...] = dinv[:, None]

    return pl.pallas_call(
        body,
        grid=(n // mb,),
        in_specs=[pl.BlockSpec((mb, p), lambda i: (i, 0)),
                  pl.BlockSpec((mb, h), lambda i: (i, 0))],
        out_specs=[pl.BlockSpec((mb, h), lambda i: (i, 0)),
                   pl.BlockSpec((mb, 1), lambda i: (i, 0))],
        out_shape=[jax.ShapeDtypeStruct((n, h), jnp.float32),
                   jax.ShapeDtypeStruct((n, 1), jnp.float32)],
    )(degp_t, xw)


def _layer_mm(sp, y, dinv, b, w):
    _, n, h = sp.shape
    mb = 1000

    def body(sp_ref, y_ref, di_ref, b_ref, w_ref, o_ref):
        t = di_ref[...] * (sp_ref[0] + sp_ref[1] + y_ref[...]) + b_ref[...]
        hh = jnp.maximum(t, 0.0)
        o_ref[...] = jnp.dot(hh, w_ref[...],
                             preferred_element_type=jnp.float32) * di_ref[...]

    return pl.pallas_call(
        body,
        grid=(n // mb,),
        in_specs=[pl.BlockSpec((2, mb, h), lambda i: (0, i, 0)),
                  pl.BlockSpec((mb, h), lambda i: (i, 0)),
                  pl.BlockSpec((mb, 1), lambda i: (i, 0)),
                  pl.BlockSpec((h,), lambda i: (0,)),
                  pl.BlockSpec((h, h), lambda i: (0, 0))],
        out_specs=pl.BlockSpec((mb, h), lambda i: (i, 0)),
        out_shape=jax.ShapeDtypeStruct((n, h), jnp.float32),
    )(sp, y, dinv, b, w)


def _layer_mm_final(sp, y, dinv, b, wcat, bl1):
    _, n, h = sp.shape
    mb = 1000

    def body(sp_ref, y_ref, di_ref, b_ref, w_ref, bl1_ref, a_ref, bt_ref):
        t = di_ref[...] * (sp_ref[0] + sp_ref[1] + y_ref[...]) + b_ref[...]
        hh = jnp.maximum(t, 0.0)
        acc = jnp.dot(hh, w_ref[...], preferred_element_type=jnp.float32)
        a_ref[...] = acc[:, :h] + bl1_ref[...]
        bt_ref[...] = acc[:, h:]

    return pl.pallas_call(
        body,
        grid=(n // mb,),
        in_specs=[pl.BlockSpec((2, mb, h), lambda i: (0, i, 0)),
                  pl.BlockSpec((mb, h), lambda i: (i, 0)),
                  pl.BlockSpec((mb, 1), lambda i: (i, 0)),
                  pl.BlockSpec((h,), lambda i: (0,)),
                  pl.BlockSpec((h, 2 * h), lambda i: (0, 0)),
                  pl.BlockSpec((h,), lambda i: (0,))],
        out_specs=[pl.BlockSpec((mb, h), lambda i: (i, 0)),
                   pl.BlockSpec((mb, h), lambda i: (i, 0))],
        out_shape=[jax.ShapeDtypeStruct((n, h), jnp.float32),
                   jax.ShapeDtypeStruct((n, h), jnp.float32)],
    )(sp, y, dinv, b, wcat, bl1)


def _final(z, wl2, bl2):
    e, h = z.shape
    c = wl2.shape[1]
    mb = 4000

    def body(z_ref, w_ref, b_ref, o_ref):
        zz = jnp.maximum(z_ref[...], 0.0)
        l = jnp.dot(zz, w_ref[...], preferred_element_type=jnp.float32) + b_ref[...]
        m = jnp.max(l, axis=1, keepdims=True)
        s = l - m
        lse = jnp.log(jnp.sum(jnp.exp(s), axis=1, keepdims=True))
        o_ref[...] = s - lse

    return pl.pallas_call(
        body,
        grid=(e // mb,),
        in_specs=[pl.BlockSpec((mb, h), lambda i: (i, 0)),
                  pl.BlockSpec((h, c), lambda i: (0, 0)),
                  pl.BlockSpec((c,), lambda i: (0,))],
        out_specs=pl.BlockSpec((mb, c), lambda i: (i, 0)),
        out_shape=jax.ShapeDtypeStruct((e, c), jnp.float32),
    )(z, wl2, bl2)


def _sc_degree(dst, n):
    e = dst.shape[0]
    epw = e // NW
    full = epw // 16
    rem = epw - full * 16
    mesh = plsc.VectorSubcoreMesh(core_axis_name="c", subcore_axis_name="s")

    @functools.partial(
        pl.kernel,
        out_type=jax.ShapeDtypeStruct((NW, n), jnp.float32),
        mesh=mesh,
        compiler_params=_SC_PARAMS,
        scratch_types=[
            pltpu.VMEM((epw + 16,), jnp.int32),
            pltpu.VMEM((n,), jnp.float32),
        ],
    )
    def k(dst_hbm, out_hbm, idx_v, hist_v):
        cid = lax.axis_index("c")
        sid = lax.axis_index("s")
        wid = sid * 2 + cid
        zeros16 = jnp.zeros((16,), jnp.float32)
        ones16 = jnp.ones((16,), jnp.float32)

        def zero_body(i, _):
            hist_v[pl.ds(i * 16, 16)] = zeros16
            return 0
        lax.fori_loop(0, n // 16, zero_body, 0)

        pltpu.sync_copy(dst_hbm.at[pl.ds(wid * epw, epw)], idx_v.at[pl.ds(0, epw)])

        def body(i, _):
            v = idx_v[pl.ds(i * 16, 16)]
            plsc.addupdate_scatter(hist_v, [v], ones16)
            return 0
        lax.fori_loop(0, full, body, 0)
        if rem:
            v = idx_v[pl.ds(full * 16, 16)]
            mask = lax.iota(jnp.int32, 16) < rem
            plsc.addupdate_scatter(hist_v, [v], ones16, mask=mask)

        pltpu.sync_copy(hist_v, out_hbm.at[wid])

    return k(dst)


def _sc_aggregate(y, src, dst, zeros):
    n, h = y.shape
    e = src.shape[0]
    epw = e // NW
    nfull = epw // CHUNK
    tail = epw - nfull * CHUNK
    rps = (n // (16 * 8)) * 8
    rextra = n - 16 * rps
    mesh = plsc.VectorSubcoreMesh(core_axis_name="c", subcore_axis_name="s")

    npairs = nfull // 2
    leftover = nfull - 2 * npairs

    @functools.partial(
        pl.kernel,
        out_type=jax.ShapeDtypeStruct((2, n, h), jnp.float32),
        mesh=mesh,
        compiler_params=_SC_PARAMS,
        scratch_types=[
            pltpu.VMEM((epw + 16,), jnp.int32),
            pltpu.VMEM((epw + 16,), jnp.int32),
            pltpu.VMEM((CHUNK,), jnp.int32),
            pltpu.VMEM((CHUNK,), jnp.int32),
            pltpu.VMEM((CHUNK,), jnp.int32),
            pltpu.VMEM((CHUNK,), jnp.int32),
            pltpu.VMEM((CHUNK, h), jnp.float32),
            pltpu.VMEM((CHUNK, h), jnp.float32),
            pltpu.VMEM((max(tail, 1),), jnp.int32),
            pltpu.VMEM((max(tail, 1),), jnp.int32),
            pltpu.VMEM((max(tail, 1), h), jnp.float32),
            pltpu.VMEM_SHARED((n, h), jnp.float32),
            pltpu.SemaphoreType.DMA,
            pltpu.SemaphoreType.DMA,
            pltpu.SemaphoreType.DMA,
            pltpu.SemaphoreType.DMA,
        ],
    )
    def k(y_hbm, src_hbm, dst_hbm, zero_hbm, out_hbm,
          sall, dall, si0, si1, di0, di1, rows0, rows1,
          sidx_t, didx_t, rows_t, stab,
          semg0, semg1, sems0, sems1):
        cid = lax.axis_index("c")
        sid = lax.axis_index("s")
        wid = sid * 2 + cid
        base = wid * epw
        r0 = pl.multiple_of(sid * rps, 8)
        pltpu.sync_copy(zero_hbm.at[pl.ds(r0, rps)], stab.at[pl.ds(r0, rps)])
        if rextra:
            @pl.when(sid == 15)
            def _():
                pltpu.sync_copy(zero_hbm.at[pl.ds(16 * rps, rextra)],
                                stab.at[pl.ds(16 * rps, rextra)])
        pltpu.sync_copy(src_hbm.at[pl.ds(base, epw)], sall.at[pl.ds(0, epw)])
        pltpu.sync_copy(dst_hbm.at[pl.ds(base, epw)], dall.at[pl.ds(0, epw)])
        plsc.subcore_barrier()

        def stage(j, buf_all, buf_idx, m):
            for c in range(m // 16):
                buf_idx[pl.ds(c * 16, 16)] = buf_all[pl.ds(j * CHUNK + c * 16, 16)]

        def pair(t, _):
            a = 2 * t
            b = a + 1
            stage(a, sall, si0, CHUNK)
            stage(a, dall, di0, CHUNK)
            stage(b, sall, si1, CHUNK)
            stage(b, dall, di1, CHUNK)
            ga = pltpu.async_copy(y_hbm.at[si0], rows0, semg0)
            gb = pltpu.async_copy(y_hbm.at[si1], rows1, semg1)
            ga.wait()
            sa = pltpu.async_copy(rows0, stab.at[di0], sems0, add=True)
            gb.wait()
            sb = pltpu.async_copy(rows1, stab.at[di1], sems1, add=True)
            sa.wait()
            sb.wait()
            return 0
        lax.fori_loop(0, npairs, pair, 0)

        if leftover:
            j = 2 * npairs
            stage(j, sall, si0, CHUNK)
            stage(j, dall, di0, CHUNK)
            pltpu.async_copy(y_hbm.at[si0], rows0, semg0).wait()
            pltpu.async_copy(rows0, stab.at[di0], sems0, add=True).wait()
        if tail:
            off = base + nfull * CHUNK
            pltpu.sync_copy(src_hbm.at[pl.ds(off, tail)], sidx_t)
            pltpu.sync_copy(dst_hbm.at[pl.ds(off, tail)], didx_t)
            pltpu.async_copy(y_hbm.at[sidx_t], rows_t, semg1).wait()
            pltpu.async_copy(rows_t, stab.at[didx_t], sems1, add=True).wait()

        plsc.subcore_barrier()
        pltpu.sync_copy(stab.at[pl.ds(r0, rps)],
                        out_hbm.at[cid, pl.ds(r0, rps)])
        if rextra:
            @pl.when(sid == 15)
            def _():
                pltpu.sync_copy(stab.at[pl.ds(16 * rps, rextra)],
                                out_hbm.at[cid, pl.ds(16 * rps, rextra)])

    return k(y, src, dst, zeros)


def _sc_edge(a_t, b_t, src, dst):
    n, h = a_t.shape
    e = src.shape[0]
    epw = e // NW
    nfull = epw // CHUNK
    tail = epw - nfull * CHUNK
    mesh = plsc.VectorSubcoreMesh(core_axis_name="c", subcore_axis_name="s")

    npairs = nfull // 2
    leftover = nfull - 2 * npairs

    @functools.partial(
        pl.kernel,
        out_type=jax.ShapeDtypeStruct((e, h), jnp.float32),
        mesh=mesh,
        compiler_params=_SC_PARAMS,
        scratch_types=[
            pltpu.VMEM((epw + 16,), jnp.int32),
            pltpu.VMEM((epw + 16,), jnp.int32),
            pltpu.VMEM((CHUNK,), jnp.int32),
            pltpu.VMEM((CHUNK,), jnp.int32),
            pltpu.VMEM((CHUNK,), jnp.int32),
            pltpu.VMEM((CHUNK,), jnp.int32),
            pltpu.VMEM((CHUNK, h), jnp.float32),
            pltpu.VMEM((CHUNK, h), jnp.float32),
            pltpu.VMEM((CHUNK, h), jnp.float32),
            pltpu.VMEM((CHUNK, h), jnp.float32),
            pltpu.VMEM((max(tail, 1),), jnp.int32),
            pltpu.VMEM((max(tail, 1),), jnp.int32),
            pltpu.VMEM((max(tail, 1), h), jnp.float32),
            pltpu.VMEM((max(tail, 1), h), jnp.float32),
            pltpu.SemaphoreType.DMA,
            pltpu.SemaphoreType.DMA,
            pltpu.SemaphoreType.DMA,
            pltpu.SemaphoreType.DMA,
            pltpu.SemaphoreType.DMA,
            pltpu.SemaphoreType.DMA,
        ],
    )
    def k(a_hbm, b_hbm, src_hbm, dst_hbm, z_hbm,
          sall, dall, si0, di0, si1, di1, ra0, rb0, ra1, rb1,
          sidx_t, didx_t, ra_t, rb_t,
          sga0, sgb0, sga1, sgb1, sw0, sw1):
        cid = lax.axis_index("c")
        sid = lax.axis_index("s")
        wid = sid * 2 + cid
        base = wid * epw
        pltpu.sync_copy(src_hbm.at[pl.ds(base, epw)], sall.at[pl.ds(0, epw)])
        pltpu.sync_copy(dst_hbm.at[pl.ds(base, epw)], dall.at[pl.ds(0, epw)])

        def stage(j, buf_all, buf_idx):
            for c in range(CHUNK // 16):
                buf_idx[pl.ds(c * 16, 16)] = buf_all[pl.ds(j * CHUNK + c * 16, 16)]

        def addrows(va, vb, m):
            def addrow(r, _):
                for c in range(h // 16):
                    plsc.addupdate(va.at[r, pl.ds(c * 16, 16)],
                                   vb[r, pl.ds(c * 16, 16)])
                return 0
            lax.fori_loop(0, m, addrow, 0)

        def pair(t, _):
            a = 2 * t
            b = a + 1
            offa = pl.multiple_of(base + a * CHUNK, 8)
            offb = pl.multiple_of(base + b * CHUNK, 8)
            stage(a, sall, si0)
            stage(a, dall, di0)
            stage(b, sall, si1)
            stage(b, dall, di1)
            ga = pltpu.async_copy(a_hbm.at[si0], ra0, sga0)
            gb = pltpu.async_copy(b_hbm.at[di0], rb0, sgb0)
            ga1c = pltpu.async_copy(a_hbm.at[si1], ra1, sga1)
            gb1c = pltpu.async_copy(b_hbm.at[di1], rb1, sgb1)
            ga.wait()
            gb.wait()
            addrows(ra0, rb0, CHUNK)
            wa = pltpu.async_copy(ra0, z_hbm.at[pl.ds(offa, CHUNK)], sw0)
            ga1c.wait()
            gb1c.wait()
            addrows(ra1, rb1, CHUNK)
            wb = pltpu.async_copy(ra1, z_hbm.at[pl.ds(offb, CHUNK)], sw1)
            wa.wait()
            wb.wait()
            return 0
        lax.fori_loop(0, npairs, pair, 0)

        if leftover:
            j = 2 * npairs
            off = pl.multiple_of(base + j * CHUNK, 8)
            stage(j, sall, si0)
            stage(j, dall, di0)
            ga = pltpu.async_copy(a_hbm.at[si0], ra0, sga0)
            gb = pltpu.async_copy(b_hbm.at[di0], rb0, sgb0)
            ga.wait()
            gb.wait()
            addrows(ra0, rb0, CHUNK)
            pltpu.async_copy(ra0, z_hbm.at[pl.ds(off, CHUNK)], sw0).wait()
        if tail:
            off = base + nfull * CHUNK
            pltpu.sync_copy(src_hbm.at[pl.ds(off, tail)], sidx_t)
            pltpu.sync_copy(dst_hbm.at[pl.ds(off, tail)], didx_t)
            ga = pltpu.async_copy(a_hbm.at[sidx_t], ra_t, sga1)
            gb = pltpu.async_copy(b_hbm.at[didx_t], rb_t, sgb1)
            ga.wait()
            gb.wait()
            addrows(ra_t, rb_t, tail)
            pltpu.async_copy(ra_t, z_hbm.at[pl.ds(pl.multiple_of(off, 8), tail)],
                             sw1).wait()

    return k(a_t, b_t, src, dst)


def kernel(x, edge_index, W1, b1, W2, b2, Wl1, bl1, Wl2, bl2):
    n, _ = x.shape
    h = W1.shape[1]
    src = edge_index[0]
    dst = edge_index[1]

    xw1 = _matmul(x, W1, 1000)
    degp = _sc_degree(dst, n)
    y1, dinv = _dinv_scale(degp.T, xw1)
    zeros = jnp.zeros((n, h), jnp.float32)
    sp1 = _sc_aggregate(y1, src, dst, zeros)
    y2 = _layer_mm(sp1, y1, dinv, b1, W2)
    sp2 = _sc_aggregate(y2, src, dst, zeros)
    wcat = jnp.concatenate([Wl1[:h], Wl1[h:]], axis=1)
    a_t, b_t = _layer_mm_final(sp2, y2, dinv, b2, wcat, bl1)
    z = _sc_edge(a_t, b_t, src, dst)
    return _final(z, Wl2, bl2)


# speedup vs baseline: 10.6798x; 1.0045x over previous
import functools

import jax
import jax.numpy as jnp
from jax import lax
from jax.experimental import pallas as pl
from jax.experimental.pallas import tpu as pltpu
from jax.experimental.pallas import tpu_sc as plsc

NW = 32
CHUNK = 128
_SC_PARAMS = pltpu.CompilerParams(needs_layout_passes=False)


def _matmul(x, w, mb):
    m, k = x.shape
    _, n = w.shape

    def body(x_ref, w_ref, o_ref):
        o_ref[...] = jnp.dot(x_ref[...], w_ref[...],
                             preferred_element_type=jnp.float32)

    return pl.pallas_call(
        body,
        grid=(m // mb,),
        in_specs=[pl.BlockSpec((mb, k), lambda i: (i, 0)),
                  pl.BlockSpec((k, n), lambda i: (0, 0))],
        out_specs=pl.BlockSpec((mb, n), lambda i: (i, 0)),
        out_shape=jax.ShapeDtypeStruct((m, n), jnp.float32),
    )(x, w)


def _matmul_scale(x, w, degp_t, mb):
    m, k = x.shape
    _, n = w.shape
    p = degp_t.shape[1]

    def body(x_ref, w_ref, d_ref, y_ref, dinv_ref):
        deg = jnp.sum(d_ref[...], axis=1) + 1.0
        dinv = lax.rsqrt(deg)
        acc = jnp.dot(x_ref[...], w_ref[...], preferred_element_type=jnp.float32)
        y_ref[...] = acc * dinv[:, None]
        dinv_ref[...] = dinv[:, None]

    return pl.pallas_call(
        body,
        grid=(m // mb,),
        in_specs=[pl.BlockSpec((mb, k), lambda i: (i, 0)),
                  pl.BlockSpec((k, n), lambda i: (0, 0)),
                  pl.BlockSpec((mb, p), lambda i: (i, 0))],
        out_specs=[pl.BlockSpec((mb, n), lambda i: (i, 0)),
                   pl.BlockSpec((mb, 1), lambda i: (i, 0))],
        out_shape=[jax.ShapeDtypeStruct((m, n), jnp.float32),
                   jax.ShapeDtypeStruct((m, 1), jnp.float32)],
    )(x, w, degp_t)


def _layer_mm(sp, y, dinv, b, w):
    _, n, h = sp.shape
    mb = 1000

    def body(sp_ref, y_ref, di_ref, b_ref, w_ref, o_ref):
        t = di_ref[...] * (sp_ref[0] + sp_ref[1] + y_ref[...]) + b_ref[...]
        hh = jnp.maximum(t, 0.0)
        o_ref[...] = jnp.dot(hh, w_ref[...],
                             preferred_element_type=jnp.float32) * di_ref[...]

    return pl.pallas_call(
        body,
        grid=(n // mb,),
        in_specs=[pl.BlockSpec((2, mb, h), lambda i: (0, i, 0)),
                  pl.BlockSpec((mb, h), lambda i: (i, 0)),
                  pl.BlockSpec((mb, 1), lambda i: (i, 0)),
                  pl.BlockSpec((h,), lambda i: (0,)),
                  pl.BlockSpec((h, h), lambda i: (0, 0))],
        out_specs=pl.BlockSpec((mb, h), lambda i: (i, 0)),
        out_shape=jax.ShapeDtypeStruct((n, h), jnp.float32),
    )(sp, y, dinv, b, w)


def _layer_mm_final(sp, y, dinv, b, wcat, bl1):
    _, n, h = sp.shape
    mb = 1000

    def body(sp_ref, y_ref, di_ref, b_ref, w_ref, bl1_ref, a_ref, bt_ref):
        t = di_ref[...] * (sp_ref[0] + sp_ref[1] + y_ref[...]) + b_ref[...]
        hh = jnp.maximum(t, 0.0)
        acc = jnp.dot(hh, w_ref[...], preferred_element_type=jnp.float32)
        a_ref[...] = acc[:, :h] + bl1_ref[...]
        bt_ref[...] = acc[:, h:]

    return pl.pallas_call(
        body,
        grid=(n // mb,),
        in_specs=[pl.BlockSpec((2, mb, h), lambda i: (0, i, 0)),
                  pl.BlockSpec((mb, h), lambda i: (i, 0)),
                  pl.BlockSpec((mb, 1), lambda i: (i, 0)),
                  pl.BlockSpec((h,), lambda i: (0,)),
                  pl.BlockSpec((h, 2 * h), lambda i: (0, 0)),
                  pl.BlockSpec((h,), lambda i: (0,))],
        out_specs=[pl.BlockSpec((mb, h), lambda i: (i, 0)),
                   pl.BlockSpec((mb, h), lambda i: (i, 0))],
        out_shape=[jax.ShapeDtypeStruct((n, h), jnp.float32),
                   jax.ShapeDtypeStruct((n, h), jnp.float32)],
    )(sp, y, dinv, b, wcat, bl1)


def _final(z, wl2, bl2):
    e, h = z.shape
    c = wl2.shape[1]
    mb = 4000

    def body(z_ref, w_ref, b_ref, o_ref):
        zz = jnp.maximum(z_ref[...], 0.0)
        l = jnp.dot(zz, w_ref[...], preferred_element_type=jnp.float32) + b_ref[...]
        m = jnp.max(l, axis=1, keepdims=True)
        s = l - m
        lse = jnp.log(jnp.sum(jnp.exp(s), axis=1, keepdims=True))
        o_ref[...] = s - lse

    return pl.pallas_call(
        body,
        grid=(e // mb,),
        in_specs=[pl.BlockSpec((mb, h), lambda i: (i, 0)),
                  pl.BlockSpec((h, c), lambda i: (0, 0)),
                  pl.BlockSpec((c,), lambda i: (0,))],
        out_specs=pl.BlockSpec((mb, c), lambda i: (i, 0)),
        out_shape=jax.ShapeDtypeStruct((e, c), jnp.float32),
    )(z, wl2, bl2)


def _sc_degree(dst, n):
    e = dst.shape[0]
    epw = e // NW
    full = epw // 16
    rem = epw - full * 16
    mesh = plsc.VectorSubcoreMesh(core_axis_name="c", subcore_axis_name="s")

    @functools.partial(
        pl.kernel,
        out_type=jax.ShapeDtypeStruct((NW, n), jnp.float32),
        mesh=mesh,
        compiler_params=_SC_PARAMS,
        scratch_types=[
            pltpu.VMEM((epw + 16,), jnp.int32),
            pltpu.VMEM((n,), jnp.float32),
        ],
    )
    def k(dst_hbm, out_hbm, idx_v, hist_v):
        cid = lax.axis_index("c")
        sid = lax.axis_index("s")
        wid = sid * 2 + cid
        zeros16 = jnp.zeros((16,), jnp.float32)
        ones16 = jnp.ones((16,), jnp.float32)

        def zero_body(i, _):
            hist_v[pl.ds(i * 16, 16)] = zeros16
            return 0
        lax.fori_loop(0, n // 16, zero_body, 0)

        pltpu.sync_copy(dst_hbm.at[pl.ds(wid * epw, epw)], idx_v.at[pl.ds(0, epw)])

        def body(i, _):
            v = idx_v[pl.ds(i * 16, 16)]
            plsc.addupdate_scatter(hist_v, [v], ones16)
            return 0
        lax.fori_loop(0, full, body, 0)
        if rem:
            v = idx_v[pl.ds(full * 16, 16)]
            mask = lax.iota(jnp.int32, 16) < rem
            plsc.addupdate_scatter(hist_v, [v], ones16, mask=mask)

        pltpu.sync_copy(hist_v, out_hbm.at[wid])

    return k(dst)


def _sc_aggregate(y, src, dst, zeros):
    n, h = y.shape
    e = src.shape[0]
    epw = e // NW
    nfull = epw // CHUNK
    tail = epw - nfull * CHUNK
    rps = (n // (16 * 8)) * 8
    rextra = n - 16 * rps
    mesh = plsc.VectorSubcoreMesh(core_axis_name="c", subcore_axis_name="s")

    npairs = nfull // 2
    leftover = nfull - 2 * npairs

    @functools.partial(
        pl.kernel,
        out_type=jax.ShapeDtypeStruct((2, n, h), jnp.float32),
        mesh=mesh,
        compiler_params=_SC_PARAMS,
        scratch_types=[
            pltpu.VMEM((epw + 16,), jnp.int32),
            pltpu.VMEM((epw + 16,), jnp.int32),
            pltpu.VMEM((CHUNK,), jnp.int32),
            pltpu.VMEM((CHUNK,), jnp.int32),
            pltpu.VMEM((CHUNK,), jnp.int32),
            pltpu.VMEM((CHUNK,), jnp.int32),
            pltpu.VMEM((CHUNK, h), jnp.float32),
            pltpu.VMEM((CHUNK, h), jnp.float32),
            pltpu.VMEM((max(tail, 1),), jnp.int32),
            pltpu.VMEM((max(tail, 1),), jnp.int32),
            pltpu.VMEM((max(tail, 1), h), jnp.float32),
            pltpu.VMEM_SHARED((n, h), jnp.float32),
            pltpu.SemaphoreType.DMA,
            pltpu.SemaphoreType.DMA,
            pltpu.SemaphoreType.DMA,
            pltpu.SemaphoreType.DMA,
        ],
    )
    def k(y_hbm, src_hbm, dst_hbm, zero_hbm, out_hbm,
          sall, dall, si0, si1, di0, di1, rows0, rows1,
          sidx_t, didx_t, rows_t, stab,
          semg0, semg1, sems0, sems1):
        cid = lax.axis_index("c")
        sid = lax.axis_index("s")
        wid = sid * 2 + cid
        base = wid * epw
        r0 = pl.multiple_of(sid * rps, 8)
        pltpu.sync_copy(zero_hbm.at[pl.ds(r0, rps)], stab.at[pl.ds(r0, rps)])
        if rextra:
            @pl.when(sid == 15)
            def _():
                pltpu.sync_copy(zero_hbm.at[pl.ds(16 * rps, rextra)],
                                stab.at[pl.ds(16 * rps, rextra)])
        pltpu.sync_copy(src_hbm.at[pl.ds(base, epw)], sall.at[pl.ds(0, epw)])
        pltpu.sync_copy(dst_hbm.at[pl.ds(base, epw)], dall.at[pl.ds(0, epw)])
        plsc.subcore_barrier()

        def stage(j, buf_all, buf_idx, m):
            for c in range(m // 16):
                buf_idx[pl.ds(c * 16, 16)] = buf_all[pl.ds(j * CHUNK + c * 16, 16)]

        def pair(t, _):
            a = 2 * t
            b = a + 1
            stage(a, sall, si0, CHUNK)
            stage(a, dall, di0, CHUNK)
            stage(b, sall, si1, CHUNK)
            stage(b, dall, di1, CHUNK)
            ga = pltpu.async_copy(y_hbm.at[si0], rows0, semg0)
            gb = pltpu.async_copy(y_hbm.at[si1], rows1, semg1)
            ga.wait()
            sa = pltpu.async_copy(rows0, stab.at[di0], sems0, add=True)
            gb.wait()
            sb = pltpu.async_copy(rows1, stab.at[di1], sems1, add=True)
            sa.wait()
            sb.wait()
            return 0
        lax.fori_loop(0, npairs, pair, 0)

        if leftover:
            j = 2 * npairs
            stage(j, sall, si0, CHUNK)
            stage(j, dall, di0, CHUNK)
            pltpu.async_copy(y_hbm.at[si0], rows0, semg0).wait()
            pltpu.async_copy(rows0, stab.at[di0], sems0, add=True).wait()
        if tail:
            off = base + nfull * CHUNK
            pltpu.sync_copy(src_hbm.at[pl.ds(off, tail)], sidx_t)
            pltpu.sync_copy(dst_hbm.at[pl.ds(off, tail)], didx_t)
            pltpu.async_copy(y_hbm.at[sidx_t], rows_t, semg1).wait()
            pltpu.async_copy(rows_t, stab.at[didx_t], sems1, add=True).wait()

        plsc.subcore_barrier()
        pltpu.sync_copy(stab.at[pl.ds(r0, rps)],
                        out_hbm.at[cid, pl.ds(r0, rps)])
        if rextra:
            @pl.when(sid == 15)
            def _():
                pltpu.sync_copy(stab.at[pl.ds(16 * rps, rextra)],
                                out_hbm.at[cid, pl.ds(16 * rps, rextra)])

    return k(y, src, dst, zeros)


def _sc_edge(a_t, b_t, src, dst):
    n, h = a_t.shape
    e = src.shape[0]
    epw = e // NW
    nfull = epw // CHUNK
    tail = epw - nfull * CHUNK
    mesh = plsc.VectorSubcoreMesh(core_axis_name="c", subcore_axis_name="s")

    npairs = nfull // 2
    leftover = nfull - 2 * npairs

    @functools.partial(
        pl.kernel,
        out_type=jax.ShapeDtypeStruct((e, h), jnp.float32),
        mesh=mesh,
        compiler_params=_SC_PARAMS,
        scratch_types=[
            pltpu.VMEM((epw + 16,), jnp.int32),
            pltpu.VMEM((epw + 16,), jnp.int32),
            pltpu.VMEM((CHUNK,), jnp.int32),
            pltpu.VMEM((CHUNK,), jnp.int32),
            pltpu.VMEM((CHUNK,), jnp.int32),
            pltpu.VMEM((CHUNK,), jnp.int32),
            pltpu.VMEM((CHUNK, h), jnp.float32),
            pltpu.VMEM((CHUNK, h), jnp.float32),
            pltpu.VMEM((CHUNK, h), jnp.float32),
            pltpu.VMEM((CHUNK, h), jnp.float32),
            pltpu.VMEM((max(tail, 1),), jnp.int32),
            pltpu.VMEM((max(tail, 1),), jnp.int32),
            pltpu.VMEM((max(tail, 1), h), jnp.float32),
            pltpu.VMEM((max(tail, 1), h), jnp.float32),
            pltpu.SemaphoreType.DMA,
            pltpu.SemaphoreType.DMA,
            pltpu.SemaphoreType.DMA,
            pltpu.SemaphoreType.DMA,
            pltpu.SemaphoreType.DMA,
            pltpu.SemaphoreType.DMA,
        ],
    )
    def k(a_hbm, b_hbm, src_hbm, dst_hbm, z_hbm,
          sall, dall, si0, di0, si1, di1, ra0, rb0, ra1, rb1,
          sidx_t, didx_t, ra_t, rb_t,
          sga0, sgb0, sga1, sgb1, sw0, sw1):
        cid = lax.axis_index("c")
        sid = lax.axis_index("s")
        wid = sid * 2 + cid
        base = wid * epw
        pltpu.sync_copy(src_hbm.at[pl.ds(base, epw)], sall.at[pl.ds(0, epw)])
        pltpu.sync_copy(dst_hbm.at[pl.ds(base, epw)], dall.at[pl.ds(0, epw)])

        def stage(j, buf_all, buf_idx):
            for c in range(CHUNK // 16):
                buf_idx[pl.ds(c * 16, 16)] = buf_all[pl.ds(j * CHUNK + c * 16, 16)]

        def addrows(va, vb, m):
            def addrow(r, _):
                for c in range(h // 16):
                    plsc.addupdate(va.at[r, pl.ds(c * 16, 16)],
                                   vb[r, pl.ds(c * 16, 16)])
                return 0
            lax.fori_loop(0, m, addrow, 0)

        def pair(t, _):
            a = 2 * t
            b = a + 1
            offa = pl.multiple_of(base + a * CHUNK, 8)
            offb = pl.multiple_of(base + b * CHUNK, 8)
            stage(a, sall, si0)
            stage(a, dall, di0)
            stage(b, sall, si1)
            stage(b, dall, di1)
            ga = pltpu.async_copy(a_hbm.at[si0], ra0, sga0)
            gb = pltpu.async_copy(b_hbm.at[di0], rb0, sgb0)
            ga1c = pltpu.async_copy(a_hbm.at[si1], ra1, sga1)
            gb1c = pltpu.async_copy(b_hbm.at[di1], rb1, sgb1)
            ga.wait()
            gb.wait()
            addrows(ra0, rb0, CHUNK)
            wa = pltpu.async_copy(ra0, z_hbm.at[pl.ds(offa, CHUNK)], sw0)
            ga1c.wait()
            gb1c.wait()
            addrows(ra1, rb1, CHUNK)
            wb = pltpu.async_copy(ra1, z_hbm.at[pl.ds(offb, CHUNK)], sw1)
            wa.wait()
            wb.wait()
            return 0
        lax.fori_loop(0, npairs, pair, 0)

        if leftover:
            j = 2 * npairs
            off = pl.multiple_of(base + j * CHUNK, 8)
            stage(j, sall, si0)
            stage(j, dall, di0)
            ga = pltpu.async_copy(a_hbm.at[si0], ra0, sga0)
            gb = pltpu.async_copy(b_hbm.at[di0], rb0, sgb0)
            ga.wait()
            gb.wait()
            addrows(ra0, rb0, CHUNK)
            pltpu.async_copy(ra0, z_hbm.at[pl.ds(off, CHUNK)], sw0).wait()
        if tail:
            off = base + nfull * CHUNK
            pltpu.sync_copy(src_hbm.at[pl.ds(off, tail)], sidx_t)
            pltpu.sync_copy(dst_hbm.at[pl.ds(off, tail)], didx_t)
            ga = pltpu.async_copy(a_hbm.at[sidx_t], ra_t, sga1)
            gb = pltpu.async_copy(b_hbm.at[didx_t], rb_t, sgb1)
            ga.wait()
            gb.wait()
            addrows(ra_t, rb_t, tail)
            pltpu.async_copy(ra_t, z_hbm.at[pl.ds(pl.multiple_of(off, 8), tail)],
                             sw1).wait()

    return k(a_t, b_t, src, dst)


def kernel(x, edge_index, W1, b1, W2, b2, Wl1, bl1, Wl2, bl2):
    n, _ = x.shape
    h = W1.shape[1]
    src = edge_index[0]
    dst = edge_index[1]

    degp = _sc_degree(dst, n)
    y1, dinv = _matmul_scale(x, W1, degp.T, 1000)
    zeros = jnp.zeros((n, h), jnp.float32)
    sp1 = _sc_aggregate(y1, src, dst, zeros)
    y2 = _layer_mm(sp1, y1, dinv, b1, W2)
    sp2 = _sc_aggregate(y2, src, dst, zeros)
    wcat = jnp.concatenate([Wl1[:h], Wl1[h:]], axis=1)
    a_t, b_t = _layer_mm_final(sp2, y2, dinv, b2, wcat, bl1)
    z = _sc_edge(a_t, b_t, src, dst)
    return _final(z, Wl2, bl2)
